# Initial kernel scaffold; baseline (speedup 1.0000x reference)
import functools

import jax
import jax.numpy as jnp
from jax import lax
from jax.experimental import pallas as pl
from jax.experimental.pallas import tpu as pltpu
from jax.experimental.pallas import tpu_sc as plsc

F32 = jnp.float32
OUT = 128
TAU = 0.8
LAM = 0.5
NC = 2
NS = 16
NW = NC * NS
CH = 128


def _cdiv(a, b):
    return (a + b - 1) // b


def _elu(x):
    return jnp.where(x > 0, x, jnp.exp(jnp.minimum(x, 0.0)) - 1.0)


@functools.lru_cache(maxsize=None)
def _sc_agg_call(n_src, n_dst, e):
    n_pad = _cdiv(n_dst, NS) * NS
    rps = n_pad // NS
    t_chunks = e // CH
    mesh = plsc.VectorSubcoreMesh(core_axis_name="c", subcore_axis_name="s")

    @functools.partial(
        pl.kernel,
        mesh=mesh,
        out_type=(
            jax.ShapeDtypeStruct((NC, n_pad, OUT), F32),
            jax.ShapeDtypeStruct((NC, n_pad, 16), F32),
        ),
        scratch_types=[
            pltpu.VMEM_SHARED((n_pad, OUT), F32),
            pltpu.VMEM_SHARED((n_pad, 16), F32),
            pltpu.VMEM((CH,), jnp.int32),
            pltpu.VMEM((CH,), jnp.int32),
            pltpu.VMEM((CH, OUT), F32),
            pltpu.VMEM((CH, 16), F32),
        ],
    )
    def k(table, src, dst, zf, zd, ones, o_feat, o_deg,
          feat_sh, deg_sh, sidx, didx, rows, ones_v):
        cid = lax.axis_index("c")
        sid = lax.axis_index("s")
        wid = sid * NC + cid
        pltpu.sync_copy(zf.at[pl.ds(sid * rps, rps)],
                        feat_sh.at[pl.ds(sid * rps, rps)])
        pltpu.sync_copy(zd.at[pl.ds(sid * rps, rps)],
                        deg_sh.at[pl.ds(sid * rps, rps)])
        pltpu.sync_copy(ones, ones_v)
        plsc.subcore_barrier()

        nj = (t_chunks - 1 - wid) // NW + 1

        @pl.loop(0, nj)
        def _(j):
            base = (wid + j * NW) * CH
            pltpu.sync_copy(src.at[pl.ds(base, CH)], sidx)
            pltpu.sync_copy(dst.at[pl.ds(base, CH)], didx)
            pltpu.sync_copy(table.at[sidx], rows)
            pltpu.sync_copy(rows, feat_sh.at[didx], add=True)
            pltpu.sync_copy(ones_v, deg_sh.at[didx], add=True)

        plsc.subcore_barrier()
        pltpu.sync_copy(feat_sh.at[pl.ds(sid * rps, rps)],
                        o_feat.at[cid, pl.ds(sid * rps, rps)])
        pltpu.sync_copy(deg_sh.at[pl.ds(sid * rps, rps)],
                        o_deg.at[cid, pl.ds(sid * rps, rps)])

    return k, n_pad


def _sc_agg(table, src, dst, n_dst):
    n_src, _ = table.shape
    e = src.shape[0]
    k, n_pad = _sc_agg_call(n_src, n_dst, e)
    zf = jnp.zeros((n_pad, OUT), F32)
    zd = jnp.zeros((n_pad, 16), F32)
    ones = jnp.ones((CH, 16), F32)
    feat, deg = k(table, src.astype(jnp.int32), dst.astype(jnp.int32),
                  zf, zd, ones)
    return feat[:, :n_dst, :], deg[:, :n_dst, :]


@functools.lru_cache(maxsize=None)
def _sc_gather_call(n_src, d, m):
    t_chunks = m // CH
    mesh = plsc.VectorSubcoreMesh(core_axis_name="c", subcore_axis_name="s")

    @functools.partial(
        pl.kernel,
        mesh=mesh,
        out_type=jax.ShapeDtypeStruct((m, d), F32),
        scratch_types=[
            pltpu.VMEM((CH,), jnp.int32),
            pltpu.VMEM((CH, d), F32),
        ],
    )
    def k(table, idx, out, idx_v, rows):
        cid = lax.axis_index("c")
        sid = lax.axis_index("s")
        wid = sid * NC + cid
        nj = (t_chunks - 1 - wid) // NW + 1

        @pl.loop(0, nj)
        def _(j):
            base = (wid + j * NW) * CH
            pltpu.sync_copy(idx.at[pl.ds(base, CH)], idx_v)
            pltpu.sync_copy(table.at[idx_v], rows)
            pltpu.sync_copy(rows, out.at[pl.ds(base, CH)])

    return k


def _sc_gather(table, idx):
    n_src, d = table.shape
    m = idx.shape[0]
    m_pad = _cdiv(m, CH) * CH
    idx = idx.astype(jnp.int32)
    if m_pad != m:
        idx = jnp.concatenate([idx, jnp.zeros((m_pad - m,), jnp.int32)])
    k = _sc_gather_call(n_src, d, m_pad)
    return k(table, idx)[:m]


def _apply_act(y, act):
    if act == "elu":
        return _elu(y)
    if act == "relu":
        return jnp.maximum(y, 0.0)
    return y


def _apply_post(y, post):
    if post == "l2":
        nrm = jnp.sqrt(jnp.sum(y * y, axis=1, keepdims=True))
        return y / (nrm + 1e-8)
    if post == "sigmoid":
        return 1.0 / (1.0 + jnp.exp(-y))
    return y


def _mm(x, W, *, deg=None, C=None, b=None, act=None, post=None, bm=512):
    parts = x.ndim == 3
    n = x.shape[1] if parts else x.shape[0]
    K = x.shape[-1]
    M = W.shape[1]
    grid = _cdiv(n, bm)
    in_specs = []
    args = []
    if parts:
        in_specs.append(pl.BlockSpec((2, bm, K), lambda i: (0, i, 0)))
    else:
        in_specs.append(pl.BlockSpec((bm, K), lambda i: (i, 0)))
    args.append(x)
    has_deg = deg is not None
    if has_deg:
        in_specs.append(pl.BlockSpec((2, bm, 16), lambda i: (0, i, 0)))
        args.append(deg)
    in_specs.append(pl.BlockSpec((K, M), lambda i: (0, 0)))
    args.append(W)
    has_b = b is not None
    if has_b:
        in_specs.append(pl.BlockSpec((M,), lambda i: (0,)))
        args.append(b)
    has_c = C is not None
    if has_c:
        in_specs.append(pl.BlockSpec((bm, M), lambda i: (i, 0)))
        args.append(C)

    def body(*refs):
        it = iter(refs)
        x_ref = next(it)
        d_ref = next(it) if has_deg else None
        w_ref = next(it)
        b_ref = next(it) if has_b else None
        c_ref = next(it) if has_c else None
        o_ref = next(it)
        if parts:
            xv = x_ref[0] + x_ref[1]
        else:
            xv = x_ref[...]
        if has_deg:
            dsum = d_ref[0] + d_ref[1]
            scale = 1.0 / jnp.maximum(dsum[:, 0:1], 1.0)
            xv = xv * scale
        y = jnp.dot(xv, w_ref[...], preferred_element_type=F32)
        if has_b:
            y = y + b_ref[...][None, :]
        if has_c:
            y = y + c_ref[...]
        y = _apply_act(y, act)
        y = _apply_post(y, post)
        o_ref[...] = y

    return pl.pallas_call(
        body,
        grid=(grid,),
        in_specs=in_specs,
        out_specs=pl.BlockSpec((bm, M), lambda i: (i, 0)),
        out_shape=jax.ShapeDtypeStruct((n, M), F32),
    )(*args)


def _att_logit(z, attW, attb, atta, bm=512):
    n = z.shape[0]
    grid = _cdiv(n, bm)

    def body(z_ref, w_ref, b_ref, a_ref, o_ref):
        i = pl.program_id(0)
        t = jnp.tanh(jnp.dot(z_ref[...], w_ref[...],
                             preferred_element_type=F32) + b_ref[...][None, :])
        d = jnp.sum(t * a_ref[...][None, :], axis=1, keepdims=True)
        rid = lax.broadcasted_iota(jnp.int32, (bm, 1), 0) + i * bm
        d = jnp.where(rid < n, d, 0.0)
        s = jnp.sum(d)

        @pl.when(i == 0)
        def _():
            o_ref[0, 0] = s

        @pl.when(i > 0)
        def _():
            o_ref[0, 0] = o_ref[0, 0] + s

    return pl.pallas_call(
        body,
        grid=(grid,),
        in_specs=[
            pl.BlockSpec((bm, OUT), lambda i: (i, 0)),
            pl.BlockSpec((OUT, OUT), lambda i: (0, 0)),
            pl.BlockSpec((OUT,), lambda i: (0,)),
            pl.BlockSpec((OUT,), lambda i: (0,)),
        ],
        out_specs=pl.BlockSpec(memory_space=pltpu.SMEM),
        out_shape=jax.ShapeDtypeStruct((1, 1), F32),
    )(z, attW, attb, atta)


def _axpy(ab, x, y, bm=512):
    n = x.shape[0]
    grid = _cdiv(n, bm)

    def body(ab_ref, x_ref, y_ref, o_ref):
        o_ref[...] = ab_ref[0] * x_ref[...] + ab_ref[1] * y_ref[...]

    return pl.pallas_call(
        body,
        grid=(grid,),
        in_specs=[
            pl.BlockSpec(memory_space=pltpu.SMEM),
            pl.BlockSpec((bm, OUT), lambda i: (i, 0)),
            pl.BlockSpec((bm, OUT), lambda i: (i, 0)),
        ],
        out_specs=pl.BlockSpec((bm, OUT), lambda i: (i, 0)),
        out_shape=jax.ShapeDtypeStruct((n, OUT), F32),
    )(ab, x, y)


def _contrast_stats(zm, zs, zspk, zmpk, bm=512):
    n = zm.shape[0]
    grid = _cdiv(n, bm)

    def body(zm_ref, zsf_ref, zsb_ref, zspk_ref, zmpk_ref,
             rs_ref, cs_ref, dg_ref, pr_ref, pc_ref):
        i = pl.program_id(0)
        zmb = zm_ref[...]
        zsf = zsf_ref[...]
        logits = lax.dot_general(zmb, zsf, (((1,), (1,)), ((), ())),
                                 preferred_element_type=F32) * (1.0 / TAU)
        e = jnp.exp(logits)
        rid = lax.broadcasted_iota(jnp.int32, (bm, 1), 0) + i * bm
        valid = rid < n
        rs_ref[...] = jnp.sum(e, axis=1)
        cs = jnp.sum(jnp.where(valid, e, 0.0), axis=0)

        @pl.when(i == 0)
        def _():
            cs_ref[...] = cs

        @pl.when(i > 0)
        def _():
            cs_ref[...] = cs_ref[...] + cs

        zsb = zsb_ref[...]
        dg_ref[...] = jnp.exp(jnp.sum(zmb * zsb, axis=1) * (1.0 / TAU))
        pr_ref[...] = jnp.exp(jnp.sum(zmb * zspk_ref[...], axis=1) * (1.0 / TAU))
        pc_ref[...] = jnp.exp(jnp.sum(zmpk_ref[...] * zsb, axis=1) * (1.0 / TAU))

    sd = jax.ShapeDtypeStruct((n,), F32)
    return pl.pallas_call(
        body,
        grid=(grid,),
        in_specs=[
            pl.BlockSpec((bm, OUT), lambda i: (i, 0)),
            pl.BlockSpec((n, OUT), lambda i: (0, 0)),
            pl.BlockSpec((bm, OUT), lambda i: (i, 0)),
            pl.BlockSpec((bm, OUT), lambda i: (i, 0)),
            pl.BlockSpec((bm, OUT), lambda i: (i, 0)),
        ],
        out_specs=[
            pl.BlockSpec((bm,), lambda i: (i,)),
            pl.BlockSpec((n,), lambda i: (0,)),
            pl.BlockSpec((bm,), lambda i: (i,)),
            pl.BlockSpec((bm,), lambda i: (i,)),
            pl.BlockSpec((bm,), lambda i: (i,)),
        ],
        out_shape=(sd, sd, sd, sd, sd),
    )(zm, zs, zs, zspk, zmpk)


def _contrast_loss(rs, cs, dg, pr, pc):
    n = rs.shape[0]

    def body(rs_ref, cs_ref, dg_ref, pr_ref, pc_ref, o_ref):
        l_mp = -jnp.sum(jnp.log((pr_ref[...] + dg_ref[...]) /
                                (rs_ref[...] + 1e-8) + 1e-8)) / n
        l_sc = -jnp.sum(jnp.log((pc_ref[...] + dg_ref[...]) /
                                (cs_ref[...] + 1e-8) + 1e-8)) / n
        o_ref[0, 0] = LAM * l_mp + (1.0 - LAM) * l_sc

    return pl.pallas_call(
        body,
        in_specs=[pl.BlockSpec((n,), lambda: (0,))] * 5,
        out_specs=pl.BlockSpec(memory_space=pltpu.SMEM),
        out_shape=jax.ShapeDtypeStruct((1, 1), F32),
    )(rs, cs, dg, pr, pc)


def _sc_layer(hd, rels, p):
    basis, comb, selfW = p["basis"], p["comb"], p["self"]
    out = {t: _mm(hd[t], selfW) for t in hd}
    incoming = {}
    for r, (st, dt, src, dst) in enumerate(rels):
        incoming.setdefault(dt, []).append((r, st, src, dst))
    for dt, lst in incoming.items():
        acc = out[dt]
        for j, (r, st, src, dst) in enumerate(lst):
            Wr = comb[r, 0] * basis[0] + comb[r, 1] * basis[1]
            feat, deg = _sc_agg(hd[st], src, dst, hd[dt].shape[0])
            last = j == len(lst) - 1
            acc = _mm(feat, Wr, deg=deg, C=acc,
                      act="elu" if last else None)
        out[dt] = acc
    return out


def _mp_encode(h, eis, p):
    n = h.shape[0]
    zs = []
    ws = []
    for j, ei in enumerate(eis):
        feat, deg = _sc_agg(h, ei[0], ei[1], n)
        z = _mm(feat, p["W"][j], deg=deg, act="elu")
        zs.append(z)
        w = _att_logit(z, p["attW"], p["attb"], p["atta"])[0, 0] / n
        ws.append(w)
    beta = jax.nn.softmax(jnp.stack(ws))
    return _axpy(beta, zs[0], zs[1])


def _contrast_key(z_mp, z_sc, pk, p):
    a = _mm(z_mp, p["W1"], b=p["b1"], act="elu")
    zm = _mm(a, p["W2"], b=p["b2"], post="l2")
    a = _mm(z_sc, p["W1"], b=p["b1"], act="elu")
    zsn = _mm(a, p["W2"], b=p["b2"], post="l2")
    zspk = _sc_gather(zsn, pk)
    zmpk = _sc_gather(zm, pk)
    rs, cs, dg, pr, pc = _contrast_stats(zm, zsn, zspk, zmpk)
    return _contrast_loss(rs, cs, dg, pr, pc)[0, 0]


def kernel(x_drug, x_protein, x_sideeffect, x_disease, mp_drug_0, mp_drug_1,
           mp_protein_0, mp_protein_1, edge_dp, edge_ds, edge_pd, pos_drug,
           pos_protein, dti, params):
    p = params
    xs = {"drug": x_drug, "protein": x_protein,
          "sideeffect": x_sideeffect, "disease": x_disease}
    h = {t: _mm(xs[t], p["fc"][t]["W"], b=p["fc"][t]["b"], act="elu")
         for t in xs}

    rels = [("drug", "protein", edge_dp[0], edge_dp[1]),
            ("protein", "drug", edge_dp[1], edge_dp[0]),
            ("drug", "sideeffect", edge_ds[0], edge_ds[1]),
            ("sideeffect", "drug", edge_ds[1], edge_ds[0]),
            ("protein", "disease", edge_pd[0], edge_pd[1]),
            ("disease", "protein", edge_pd[1], edge_pd[0])]

    z_sc = _sc_layer(h, rels, p["sc"])
    z_sc = _sc_layer(z_sc, rels, p["sc2"])

    mps = {"drug": [mp_drug_0, mp_drug_1],
           "protein": [mp_protein_0, mp_protein_1]}
    z_mp = {k: _mp_encode(h[k], mps[k], p["mp"][k])
            for k in ("drug", "protein")}
    z_mp = {k: _mp_encode(z_mp[k], mps[k], p["mp2"][k])
            for k in ("drug", "protein")}

    loss = (_contrast_key(z_mp["drug"], z_sc["drug"], pos_drug,
                          p["contrast"])
            + _contrast_key(z_mp["protein"], z_sc["protein"], pos_protein,
                            p["contrast"])) / 2.0

    z_d = jnp.concatenate([z_sc["drug"], z_mp["drug"]], axis=1)
    z_p = jnp.concatenate([z_sc["protein"], z_mp["protein"]], axis=1)
    rows_d = _sc_gather(z_d, dti[:, 0])
    rows_p = _sc_gather(z_p, dti[:, 1])
    H = jnp.concatenate([rows_d, rows_p], axis=1)
    h1 = _mm(H, p["pred"]["W1"], b=p["pred"]["b1"], act="relu")
    W2 = jnp.pad(p["pred"]["W2"], ((0, 0), (0, OUT - 1)))
    b2 = jnp.pad(p["pred"]["b2"], (0, OUT - 1))
    o = _mm(h1, W2, b=b2, post="sigmoid")
    return loss, o[:, :1]


# speedup vs baseline: 3.7221x; 3.7221x over previous
import functools

import jax
import jax.numpy as jnp
from jax import lax
from jax.experimental import pallas as pl
from jax.experimental.pallas import tpu as pltpu
from jax.experimental.pallas import tpu_sc as plsc

F32 = jnp.float32
OUT = 128
TAU = 0.8
LAM = 0.5
NC = 2
NS = 16
NW = NC * NS
CH = 128


def _cdiv(a, b):
    return (a + b - 1) // b


def _elu(x):
    return jnp.where(x > 0, x, jnp.exp(jnp.minimum(x, 0.0)) - 1.0)


@functools.lru_cache(maxsize=None)
def _sc_agg_call(n_src, n_dst, e):
    n_pad = _cdiv(n_dst, NS * 8) * NS * 8
    rps = n_pad // NS
    t_chunks = e // CH
    mesh = plsc.VectorSubcoreMesh(core_axis_name="c", subcore_axis_name="s")

    @functools.partial(
        pl.kernel,
        mesh=mesh,
        out_type=(
            jax.ShapeDtypeStruct((NC, n_pad, OUT), F32),
            jax.ShapeDtypeStruct((NC * n_pad,), F32),
        ),
        scratch_types=[
            pltpu.VMEM_SHARED((n_pad, OUT), F32),
            pltpu.VMEM_SHARED((n_pad,), F32),
            pltpu.VMEM((CH,), jnp.int32),
            pltpu.VMEM((CH,), jnp.int32),
            pltpu.VMEM((CH, OUT), F32),
            pltpu.VMEM((CH,), F32),
            pltpu.VMEM((rps,), F32),
            pltpu.SemaphoreType.DMA,
        ],
    )
    def k(table, src, dst, zf, zd, ones, o_feat, o_deg,
          feat_sh, deg_sh, sidx, didx, rows, ones_v, dbuf, sem):
        cid = lax.axis_index("c")
        sid = lax.axis_index("s")
        wid = sid * NC + cid
        pltpu.sync_copy(zf.at[pl.ds(sid * rps, rps)],
                        feat_sh.at[pl.ds(sid * rps, rps)])
        pltpu.sync_copy(zd.at[pl.ds(sid * rps, rps)], dbuf)
        pltpu.sync_copy(dbuf, deg_sh.at[pl.ds(sid * rps, rps)])
        pltpu.sync_copy(ones, ones_v)
        plsc.subcore_barrier()

        nj = (t_chunks - 1 - wid) // NW + 1

        @pl.loop(0, nj)
        def _(j):
            base = (wid + j * NW) * CH
            pltpu.sync_copy(src.at[pl.ds(base, CH)], sidx)
            pltpu.sync_copy(dst.at[pl.ds(base, CH)], didx)
            pltpu.async_copy(table.at[sidx], rows, sem).wait()
            pltpu.sync_copy(rows, feat_sh.at[didx], add=True)
            pltpu.sync_copy(ones_v, deg_sh.at[didx], add=True)

        plsc.subcore_barrier()
        pltpu.sync_copy(feat_sh.at[pl.ds(sid * rps, rps)],
                        o_feat.at[cid, pl.ds(sid * rps, rps)])
        pltpu.sync_copy(deg_sh.at[pl.ds(sid * rps, rps)], dbuf)
        pltpu.sync_copy(dbuf, o_deg.at[pl.ds(cid * n_pad + sid * rps, rps)])

    return k, n_pad


def _sc_agg(table, src, dst, n_dst):
    n_src, _ = table.shape
    e = src.shape[0]
    k, n_pad = _sc_agg_call(n_src, n_dst, e)
    zf = jnp.zeros((n_pad, OUT), F32)
    zd = jnp.zeros((n_pad,), F32)
    ones = jnp.ones((CH,), F32)
    feat, deg = k(table, src.astype(jnp.int32), dst.astype(jnp.int32),
                  zf, zd, ones)
    return feat[:, :n_dst, :], deg.reshape(NC, n_pad)[:, :n_dst]


@functools.lru_cache(maxsize=None)
def _sc_gather_call(n_src, d, m):
    t_chunks = m // CH
    mesh = plsc.VectorSubcoreMesh(core_axis_name="c", subcore_axis_name="s")

    @functools.partial(
        pl.kernel,
        mesh=mesh,
        out_type=jax.ShapeDtypeStruct((m, d), F32),
        scratch_types=[
            pltpu.VMEM((CH,), jnp.int32),
            pltpu.VMEM((CH, d), F32),
        ],
    )
    def k(table, idx, out, idx_v, rows):
        cid = lax.axis_index("c")
        sid = lax.axis_index("s")
        wid = sid * NC + cid
        nj = (t_chunks - 1 - wid) // NW + 1

        @pl.loop(0, nj)
        def _(j):
            base = (wid + j * NW) * CH
            pltpu.sync_copy(idx.at[pl.ds(base, CH)], idx_v)
            pltpu.sync_copy(table.at[idx_v], rows)
            pltpu.sync_copy(rows, out.at[pl.ds(base, CH)])

    return k


def _sc_gather(table, idx):
    n_src, d = table.shape
    m = idx.shape[0]
    m_pad = _cdiv(m, CH) * CH
    idx = idx.astype(jnp.int32)
    if m_pad != m:
        idx = jnp.concatenate([idx, jnp.zeros((m_pad - m,), jnp.int32)])
    k = _sc_gather_call(n_src, d, m_pad)
    return k(table, idx)[:m]


def _apply_act(y, act):
    if act == "elu":
        return _elu(y)
    if act == "relu":
        return jnp.maximum(y, 0.0)
    return y


def _apply_post(y, post):
    if post == "l2":
        nrm = jnp.sqrt(jnp.sum(y * y, axis=1, keepdims=True))
        return y / (nrm + 1e-8)
    if post == "sigmoid":
        return 1.0 / (1.0 + jnp.exp(-y))
    return y


def _mm(x, W, *, deg=None, C=None, b=None, act=None, post=None, bm=512):
    parts = x.ndim == 3
    n = x.shape[1] if parts else x.shape[0]
    K = x.shape[-1]
    M = W.shape[1]
    grid = _cdiv(n, bm)
    in_specs = []
    args = []
    if parts:
        in_specs.append(pl.BlockSpec((2, bm, K), lambda i: (0, i, 0)))
    else:
        in_specs.append(pl.BlockSpec((bm, K), lambda i: (i, 0)))
    args.append(x)
    has_deg = deg is not None
    if has_deg:
        in_specs.append(pl.BlockSpec((2, bm), lambda i: (0, i)))
        args.append(deg)
    in_specs.append(pl.BlockSpec((K, M), lambda i: (0, 0)))
    args.append(W)
    has_b = b is not None
    if has_b:
        in_specs.append(pl.BlockSpec((M,), lambda i: (0,)))
        args.append(b)
    has_c = C is not None
    if has_c:
        in_specs.append(pl.BlockSpec((bm, M), lambda i: (i, 0)))
        args.append(C)

    def body(*refs):
        it = iter(refs)
        x_ref = next(it)
        d_ref = next(it) if has_deg else None
        w_ref = next(it)
        b_ref = next(it) if has_b else None
        c_ref = next(it) if has_c else None
        o_ref = next(it)
        if parts:
            xv = x_ref[0] + x_ref[1]
        else:
            xv = x_ref[...]
        if has_deg:
            dsum = d_ref[0] + d_ref[1]
            scale = 1.0 / jnp.maximum(dsum, 1.0)
            xv = xv * scale[:, None]
        y = jnp.dot(xv, w_ref[...], preferred_element_type=F32)
        if has_b:
            y = y + b_ref[...][None, :]
        if has_c:
            y = y + c_ref[...]
        y = _apply_act(y, act)
        y = _apply_post(y, post)
        o_ref[...] = y

    return pl.pallas_call(
        body,
        grid=(grid,),
        in_specs=in_specs,
        out_specs=pl.BlockSpec((bm, M), lambda i: (i, 0)),
        out_shape=jax.ShapeDtypeStruct((n, M), F32),
    )(*args)


def _att_logit(z, attW, attb, atta, bm=512):
    n = z.shape[0]
    grid = _cdiv(n, bm)

    def body(z_ref, w_ref, b_ref, a_ref, o_ref):
        i = pl.program_id(0)
        t = jnp.tanh(jnp.dot(z_ref[...], w_ref[...],
                             preferred_element_type=F32) + b_ref[...][None, :])
        d = jnp.sum(t * a_ref[...][None, :], axis=1, keepdims=True)
        rid = lax.broadcasted_iota(jnp.int32, (bm, 1), 0) + i * bm
        d = jnp.where(rid < n, d, 0.0)
        s = jnp.sum(d)

        @pl.when(i == 0)
        def _():
            o_ref[0, 0] = s

        @pl.when(i > 0)
        def _():
            o_ref[0, 0] = o_ref[0, 0] + s

    return pl.pallas_call(
        body,
        grid=(grid,),
        in_specs=[
            pl.BlockSpec((bm, OUT), lambda i: (i, 0)),
            pl.BlockSpec((OUT, OUT), lambda i: (0, 0)),
            pl.BlockSpec((OUT,), lambda i: (0,)),
            pl.BlockSpec((OUT,), lambda i: (0,)),
        ],
        out_specs=pl.BlockSpec(memory_space=pltpu.SMEM),
        out_shape=jax.ShapeDtypeStruct((1, 1), F32),
    )(z, attW, attb, atta)


def _axpy(ab, x, y, bm=512):
    n = x.shape[0]
    grid = _cdiv(n, bm)

    def body(ab_ref, x_ref, y_ref, o_ref):
        o_ref[...] = ab_ref[0] * x_ref[...] + ab_ref[1] * y_ref[...]

    return pl.pallas_call(
        body,
        grid=(grid,),
        in_specs=[
            pl.BlockSpec(memory_space=pltpu.SMEM),
            pl.BlockSpec((bm, OUT), lambda i: (i, 0)),
            pl.BlockSpec((bm, OUT), lambda i: (i, 0)),
        ],
        out_specs=pl.BlockSpec((bm, OUT), lambda i: (i, 0)),
        out_shape=jax.ShapeDtypeStruct((n, OUT), F32),
    )(ab, x, y)


def _contrast_stats(zm, zs, zspk, zmpk, bm=512):
    n = zm.shape[0]
    grid = _cdiv(n, bm)

    def body(zm_ref, zsf_ref, zsb_ref, zspk_ref, zmpk_ref,
             rs_ref, cs_ref, dg_ref, pr_ref, pc_ref):
        i = pl.program_id(0)
        zmb = zm_ref[...]
        zsf = zsf_ref[...]
        logits = lax.dot_general(zmb, zsf, (((1,), (1,)), ((), ())),
                                 preferred_element_type=F32) * (1.0 / TAU)
        e = jnp.exp(logits)
        rid = lax.broadcasted_iota(jnp.int32, (bm, 1), 0) + i * bm
        valid = rid < n
        rs_ref[...] = jnp.sum(e, axis=1)
        cs = jnp.sum(jnp.where(valid, e, 0.0), axis=0)

        @pl.when(i == 0)
        def _():
            cs_ref[...] = cs

        @pl.when(i > 0)
        def _():
            cs_ref[...] = cs_ref[...] + cs

        zsb = zsb_ref[...]
        dg_ref[...] = jnp.exp(jnp.sum(zmb * zsb, axis=1) * (1.0 / TAU))
        pr_ref[...] = jnp.exp(jnp.sum(zmb * zspk_ref[...], axis=1) * (1.0 / TAU))
        pc_ref[...] = jnp.exp(jnp.sum(zmpk_ref[...] * zsb, axis=1) * (1.0 / TAU))

    sd = jax.ShapeDtypeStruct((n,), F32)
    return pl.pallas_call(
        body,
        grid=(grid,),
        in_specs=[
            pl.BlockSpec((bm, OUT), lambda i: (i, 0)),
            pl.BlockSpec((n, OUT), lambda i: (0, 0)),
            pl.BlockSpec((bm, OUT), lambda i: (i, 0)),
            pl.BlockSpec((bm, OUT), lambda i: (i, 0)),
            pl.BlockSpec((bm, OUT), lambda i: (i, 0)),
        ],
        out_specs=[
            pl.BlockSpec((bm,), lambda i: (i,)),
            pl.BlockSpec((n,), lambda i: (0,)),
            pl.BlockSpec((bm,), lambda i: (i,)),
            pl.BlockSpec((bm,), lambda i: (i,)),
            pl.BlockSpec((bm,), lambda i: (i,)),
        ],
        out_shape=(sd, sd, sd, sd, sd),
    )(zm, zs, zs, zspk, zmpk)


def _contrast_loss(rs, cs, dg, pr, pc):
    n = rs.shape[0]

    def body(rs_ref, cs_ref, dg_ref, pr_ref, pc_ref, o_ref):
        l_mp = -jnp.sum(jnp.log((pr_ref[...] + dg_ref[...]) /
                                (rs_ref[...] + 1e-8) + 1e-8)) / n
        l_sc = -jnp.sum(jnp.log((pc_ref[...] + dg_ref[...]) /
                                (cs_ref[...] + 1e-8) + 1e-8)) / n
        o_ref[0, 0] = LAM * l_mp + (1.0 - LAM) * l_sc

    return pl.pallas_call(
        body,
        in_specs=[pl.BlockSpec((n,), lambda: (0,))] * 5,
        out_specs=pl.BlockSpec(memory_space=pltpu.SMEM),
        out_shape=jax.ShapeDtypeStruct((1, 1), F32),
    )(rs, cs, dg, pr, pc)


def _sc_layer(hd, rels, p):
    basis, comb, selfW = p["basis"], p["comb"], p["self"]
    out = {t: _mm(hd[t], selfW) for t in hd}
    incoming = {}
    for r, (st, dt, src, dst) in enumerate(rels):
        incoming.setdefault(dt, []).append((r, st, src, dst))
    for dt, lst in incoming.items():
        acc = out[dt]
        for j, (r, st, src, dst) in enumerate(lst):
            Wr = comb[r, 0] * basis[0] + comb[r, 1] * basis[1]
            feat, deg = _sc_agg(hd[st], src, dst, hd[dt].shape[0])
            last = j == len(lst) - 1
            acc = _mm(feat, Wr, deg=deg, C=acc,
                      act="elu" if last else None)
        out[dt] = acc
    return out


def _mp_encode(h, eis, p):
    n = h.shape[0]
    zs = []
    ws = []
    for j, ei in enumerate(eis):
        feat, deg = _sc_agg(h, ei[0], ei[1], n)
        z = _mm(feat, p["W"][j], deg=deg, act="elu")
        zs.append(z)
        w = _att_logit(z, p["attW"], p["attb"], p["atta"])[0, 0] / n
        ws.append(w)
    beta = jax.nn.softmax(jnp.stack(ws))
    return _axpy(beta, zs[0], zs[1])


def _contrast_key(z_mp, z_sc, pk, p):
    a = _mm(z_mp, p["W1"], b=p["b1"], act="elu")
    zm = _mm(a, p["W2"], b=p["b2"], post="l2")
    a = _mm(z_sc, p["W1"], b=p["b1"], act="elu")
    zsn = _mm(a, p["W2"], b=p["b2"], post="l2")
    zspk = _sc_gather(zsn, pk)
    zmpk = _sc_gather(zm, pk)
    rs, cs, dg, pr, pc = _contrast_stats(zm, zsn, zspk, zmpk)
    return _contrast_loss(rs, cs, dg, pr, pc)[0, 0]


def kernel(x_drug, x_protein, x_sideeffect, x_disease, mp_drug_0, mp_drug_1,
           mp_protein_0, mp_protein_1, edge_dp, edge_ds, edge_pd, pos_drug,
           pos_protein, dti, params):
    p = params
    xs = {"drug": x_drug, "protein": x_protein,
          "sideeffect": x_sideeffect, "disease": x_disease}
    h = {t: _mm(xs[t], p["fc"][t]["W"], b=p["fc"][t]["b"], act="elu")
         for t in xs}

    rels = [("drug", "protein", edge_dp[0], edge_dp[1]),
            ("protein", "drug", edge_dp[1], edge_dp[0]),
            ("drug", "sideeffect", edge_ds[0], edge_ds[1]),
            ("sideeffect", "drug", edge_ds[1], edge_ds[0]),
            ("protein", "disease", edge_pd[0], edge_pd[1]),
            ("disease", "protein", edge_pd[1], edge_pd[0])]

    z_sc = _sc_layer(h, rels, p["sc"])
    z_sc = _sc_layer(z_sc, rels, p["sc2"])

    mps = {"drug": [mp_drug_0, mp_drug_1],
           "protein": [mp_protein_0, mp_protein_1]}
    z_mp = {k: _mp_encode(h[k], mps[k], p["mp"][k])
            for k in ("drug", "protein")}
    z_mp = {k: _mp_encode(z_mp[k], mps[k], p["mp2"][k])
            for k in ("drug", "protein")}

    loss = (_contrast_key(z_mp["drug"], z_sc["drug"], pos_drug,
                          p["contrast"])
            + _contrast_key(z_mp["protein"], z_sc["protein"], pos_protein,
                            p["contrast"])) / 2.0

    z_d = jnp.concatenate([z_sc["drug"], z_mp["drug"]], axis=1)
    z_p = jnp.concatenate([z_sc["protein"], z_mp["protein"]], axis=1)
    rows_d = _sc_gather(z_d, dti[:, 0])
    rows_p = _sc_gather(z_p, dti[:, 1])
    H = jnp.concatenate([rows_d, rows_p], axis=1)
    h1 = _mm(H, p["pred"]["W1"], b=p["pred"]["b1"], act="relu")
    W2 = jnp.pad(p["pred"]["W2"], ((0, 0), (0, OUT - 1)))
    b2 = jnp.pad(p["pred"]["b2"], (0, OUT - 1))
    o = _mm(h1, W2, b=b2, post="sigmoid")
    return loss, o[:, :1]


# speedup vs baseline: 4.0649x; 1.0921x over previous
import functools

import jax
import jax.numpy as jnp
from jax import lax
from jax.experimental import pallas as pl
from jax.experimental.pallas import tpu as pltpu
from jax.experimental.pallas import tpu_sc as plsc

F32 = jnp.float32
OUT = 128
TAU = 0.8
LAM = 0.5
NC = 2
NS = 16
NW = NC * NS
CH = 128


def _cdiv(a, b):
    return (a + b - 1) // b


def _elu(x):
    return jnp.where(x > 0, x, jnp.exp(jnp.minimum(x, 0.0)) - 1.0)


@functools.lru_cache(maxsize=None)
def _sc_agg_call(n_src, n_dst, e):
    n_pad = _cdiv(n_dst, NS * 8) * NS * 8
    rps = n_pad // NS
    s_pad = _cdiv(n_src, NS * 8) * NS * 8
    srps = s_pad // NS
    t_chunks = e // CH
    mesh = plsc.VectorSubcoreMesh(core_axis_name="c", subcore_axis_name="s")

    @functools.partial(
        pl.kernel,
        mesh=mesh,
        out_type=(
            jax.ShapeDtypeStruct((NC, n_pad, OUT), F32),
            jax.ShapeDtypeStruct((NC * n_pad,), F32),
        ),
        scratch_types=[
            pltpu.VMEM_SHARED((s_pad, OUT), F32),
            pltpu.VMEM_SHARED((n_pad, OUT), F32),
            pltpu.VMEM_SHARED((n_pad,), F32),
            pltpu.VMEM((CH,), jnp.int32),
            pltpu.VMEM((CH,), jnp.int32),
            pltpu.VMEM((CH, OUT), F32),
            pltpu.VMEM((CH,), F32),
            pltpu.VMEM((rps,), F32),
            pltpu.SemaphoreType.DMA,
        ],
    )
    def k(table, src, dst, zf, zd, ones, o_feat, o_deg,
          tab_sh, feat_sh, deg_sh, sidx, didx, rows, ones_v, dbuf, sem):
        cid = lax.axis_index("c")
        sid = lax.axis_index("s")
        wid = sid * NC + cid
        pltpu.sync_copy(table.at[pl.ds(sid * srps, srps)],
                        tab_sh.at[pl.ds(sid * srps, srps)])
        pltpu.sync_copy(zf.at[pl.ds(sid * rps, rps)],
                        feat_sh.at[pl.ds(sid * rps, rps)])
        pltpu.sync_copy(zd.at[pl.ds(sid * rps, rps)], dbuf)
        pltpu.sync_copy(dbuf, deg_sh.at[pl.ds(sid * rps, rps)])
        pltpu.sync_copy(ones, ones_v)
        plsc.subcore_barrier()

        nj = (t_chunks - 1 - wid) // NW + 1

        @pl.loop(0, nj)
        def _(j):
            base = (wid + j * NW) * CH
            pltpu.sync_copy(src.at[pl.ds(base, CH)], sidx)
            pltpu.sync_copy(dst.at[pl.ds(base, CH)], didx)
            pltpu.async_copy(tab_sh.at[sidx], rows, sem).wait()
            pltpu.sync_copy(rows, feat_sh.at[didx], add=True)
            pltpu.sync_copy(ones_v, deg_sh.at[didx], add=True)

        plsc.subcore_barrier()
        pltpu.sync_copy(feat_sh.at[pl.ds(sid * rps, rps)],
                        o_feat.at[cid, pl.ds(sid * rps, rps)])
        pltpu.sync_copy(deg_sh.at[pl.ds(sid * rps, rps)], dbuf)
        pltpu.sync_copy(dbuf, o_deg.at[pl.ds(cid * n_pad + sid * rps, rps)])

    return k, n_pad, s_pad


def _sc_agg(table, src, dst, n_dst):
    n_src, _ = table.shape
    e = src.shape[0]
    k, n_pad, s_pad = _sc_agg_call(n_src, n_dst, e)
    tab = jnp.pad(table, ((0, s_pad - n_src), (0, 0)))
    zf = jnp.zeros((n_pad, OUT), F32)
    zd = jnp.zeros((n_pad,), F32)
    ones = jnp.ones((CH,), F32)
    feat, deg = k(tab, src.astype(jnp.int32), dst.astype(jnp.int32),
                  zf, zd, ones)
    return feat[:, :n_dst, :], deg.reshape(NC, n_pad)[:, :n_dst]


@functools.lru_cache(maxsize=None)
def _sc_gather_call(n_src, d, m):
    t_chunks = m // CH
    mesh = plsc.VectorSubcoreMesh(core_axis_name="c", subcore_axis_name="s")

    @functools.partial(
        pl.kernel,
        mesh=mesh,
        out_type=jax.ShapeDtypeStruct((m, d), F32),
        scratch_types=[
            pltpu.VMEM((CH,), jnp.int32),
            pltpu.VMEM((CH, d), F32),
        ],
    )
    def k(table, idx, out, idx_v, rows):
        cid = lax.axis_index("c")
        sid = lax.axis_index("s")
        wid = sid * NC + cid
        nj = (t_chunks - 1 - wid) // NW + 1

        @pl.loop(0, nj)
        def _(j):
            base = (wid + j * NW) * CH
            pltpu.sync_copy(idx.at[pl.ds(base, CH)], idx_v)
            pltpu.sync_copy(table.at[idx_v], rows)
            pltpu.sync_copy(rows, out.at[pl.ds(base, CH)])

    return k


def _sc_gather(table, idx):
    n_src, d = table.shape
    m = idx.shape[0]
    m_pad = _cdiv(m, CH) * CH
    idx = idx.astype(jnp.int32)
    if m_pad != m:
        idx = jnp.concatenate([idx, jnp.zeros((m_pad - m,), jnp.int32)])
    k = _sc_gather_call(n_src, d, m_pad)
    return k(table, idx)[:m]


def _apply_act(y, act):
    if act == "elu":
        return _elu(y)
    if act == "relu":
        return jnp.maximum(y, 0.0)
    return y


def _apply_post(y, post):
    if post == "l2":
        nrm = jnp.sqrt(jnp.sum(y * y, axis=1, keepdims=True))
        return y / (nrm + 1e-8)
    if post == "sigmoid":
        return 1.0 / (1.0 + jnp.exp(-y))
    return y


def _mm(x, W, *, deg=None, C=None, b=None, act=None, post=None, bm=512):
    parts = x.ndim == 3
    n = x.shape[1] if parts else x.shape[0]
    K = x.shape[-1]
    M = W.shape[1]
    grid = _cdiv(n, bm)
    in_specs = []
    args = []
    if parts:
        in_specs.append(pl.BlockSpec((2, bm, K), lambda i: (0, i, 0)))
    else:
        in_specs.append(pl.BlockSpec((bm, K), lambda i: (i, 0)))
    args.append(x)
    has_deg = deg is not None
    if has_deg:
        in_specs.append(pl.BlockSpec((2, bm), lambda i: (0, i)))
        args.append(deg)
    in_specs.append(pl.BlockSpec((K, M), lambda i: (0, 0)))
    args.append(W)
    has_b = b is not None
    if has_b:
        in_specs.append(pl.BlockSpec((M,), lambda i: (0,)))
        args.append(b)
    has_c = C is not None
    if has_c:
        in_specs.append(pl.BlockSpec((bm, M), lambda i: (i, 0)))
        args.append(C)

    def body(*refs):
        it = iter(refs)
        x_ref = next(it)
        d_ref = next(it) if has_deg else None
        w_ref = next(it)
        b_ref = next(it) if has_b else None
        c_ref = next(it) if has_c else None
        o_ref = next(it)
        if parts:
            xv = x_ref[0] + x_ref[1]
        else:
            xv = x_ref[...]
        if has_deg:
            dsum = d_ref[0] + d_ref[1]
            scale = 1.0 / jnp.maximum(dsum, 1.0)
            xv = xv * scale[:, None]
        y = jnp.dot(xv, w_ref[...], preferred_element_type=F32)
        if has_b:
            y = y + b_ref[...][None, :]
        if has_c:
            y = y + c_ref[...]
        y = _apply_act(y, act)
        y = _apply_post(y, post)
        o_ref[...] = y

    return pl.pallas_call(
        body,
        grid=(grid,),
        in_specs=in_specs,
        out_specs=pl.BlockSpec((bm, M), lambda i: (i, 0)),
        out_shape=jax.ShapeDtypeStruct((n, M), F32),
    )(*args)


def _att_logit(z, attW, attb, atta, bm=512):
    n = z.shape[0]
    grid = _cdiv(n, bm)

    def body(z_ref, w_ref, b_ref, a_ref, o_ref):
        i = pl.program_id(0)
        t = jnp.tanh(jnp.dot(z_ref[...], w_ref[...],
                             preferred_element_type=F32) + b_ref[...][None, :])
        d = jnp.sum(t * a_ref[...][None, :], axis=1, keepdims=True)
        rid = lax.broadcasted_iota(jnp.int32, (bm, 1), 0) + i * bm
        d = jnp.where(rid < n, d, 0.0)
        s = jnp.sum(d)

        @pl.when(i == 0)
        def _():
            o_ref[0, 0] = s

        @pl.when(i > 0)
        def _():
            o_ref[0, 0] = o_ref[0, 0] + s

    return pl.pallas_call(
        body,
        grid=(grid,),
        in_specs=[
            pl.BlockSpec((bm, OUT), lambda i: (i, 0)),
            pl.BlockSpec((OUT, OUT), lambda i: (0, 0)),
            pl.BlockSpec((OUT,), lambda i: (0,)),
            pl.BlockSpec((OUT,), lambda i: (0,)),
        ],
        out_specs=pl.BlockSpec(memory_space=pltpu.SMEM),
        out_shape=jax.ShapeDtypeStruct((1, 1), F32),
    )(z, attW, attb, atta)


def _axpy(ab, x, y, bm=512):
    n = x.shape[0]
    grid = _cdiv(n, bm)

    def body(ab_ref, x_ref, y_ref, o_ref):
        o_ref[...] = ab_ref[0] * x_ref[...] + ab_ref[1] * y_ref[...]

    return pl.pallas_call(
        body,
        grid=(grid,),
        in_specs=[
            pl.BlockSpec(memory_space=pltpu.SMEM),
            pl.BlockSpec((bm, OUT), lambda i: (i, 0)),
            pl.BlockSpec((bm, OUT), lambda i: (i, 0)),
        ],
        out_specs=pl.BlockSpec((bm, OUT), lambda i: (i, 0)),
        out_shape=jax.ShapeDtypeStruct((n, OUT), F32),
    )(ab, x, y)


def _contrast_stats(zm, zs, zspk, zmpk, bm=512):
    n = zm.shape[0]
    grid = _cdiv(n, bm)

    def body(zm_ref, zsf_ref, zsb_ref, zspk_ref, zmpk_ref,
             rs_ref, cs_ref, dg_ref, pr_ref, pc_ref):
        i = pl.program_id(0)
        zmb = zm_ref[...]
        zsf = zsf_ref[...]
        logits = lax.dot_general(zmb, zsf, (((1,), (1,)), ((), ())),
                                 preferred_element_type=F32) * (1.0 / TAU)
        e = jnp.exp(logits)
        rid = lax.broadcasted_iota(jnp.int32, (bm, 1), 0) + i * bm
        valid = rid < n
        rs_ref[...] = jnp.sum(e, axis=1)
        cs = jnp.sum(jnp.where(valid, e, 0.0), axis=0)

        @pl.when(i == 0)
        def _():
            cs_ref[...] = cs

        @pl.when(i > 0)
        def _():
            cs_ref[...] = cs_ref[...] + cs

        zsb = zsb_ref[...]
        dg_ref[...] = jnp.exp(jnp.sum(zmb * zsb, axis=1) * (1.0 / TAU))
        pr_ref[...] = jnp.exp(jnp.sum(zmb * zspk_ref[...], axis=1) * (1.0 / TAU))
        pc_ref[...] = jnp.exp(jnp.sum(zmpk_ref[...] * zsb, axis=1) * (1.0 / TAU))

    sd = jax.ShapeDtypeStruct((n,), F32)
    return pl.pallas_call(
        body,
        grid=(grid,),
        in_specs=[
            pl.BlockSpec((bm, OUT), lambda i: (i, 0)),
            pl.BlockSpec((n, OUT), lambda i: (0, 0)),
            pl.BlockSpec((bm, OUT), lambda i: (i, 0)),
            pl.BlockSpec((bm, OUT), lambda i: (i, 0)),
            pl.BlockSpec((bm, OUT), lambda i: (i, 0)),
        ],
        out_specs=[
            pl.BlockSpec((bm,), lambda i: (i,)),
            pl.BlockSpec((n,), lambda i: (0,)),
            pl.BlockSpec((bm,), lambda i: (i,)),
            pl.BlockSpec((bm,), lambda i: (i,)),
            pl.BlockSpec((bm,), lambda i: (i,)),
        ],
        out_shape=(sd, sd, sd, sd, sd),
    )(zm, zs, zs, zspk, zmpk)


def _contrast_loss(rs, cs, dg, pr, pc):
    n = rs.shape[0]

    def body(rs_ref, cs_ref, dg_ref, pr_ref, pc_ref, o_ref):
        l_mp = -jnp.sum(jnp.log((pr_ref[...] + dg_ref[...]) /
                                (rs_ref[...] + 1e-8) + 1e-8)) / n
        l_sc = -jnp.sum(jnp.log((pc_ref[...] + dg_ref[...]) /
                                (cs_ref[...] + 1e-8) + 1e-8)) / n
        o_ref[0, 0] = LAM * l_mp + (1.0 - LAM) * l_sc

    return pl.pallas_call(
        body,
        in_specs=[pl.BlockSpec((n,), lambda: (0,))] * 5,
        out_specs=pl.BlockSpec(memory_space=pltpu.SMEM),
        out_shape=jax.ShapeDtypeStruct((1, 1), F32),
    )(rs, cs, dg, pr, pc)


def _sc_layer(hd, rels, p):
    basis, comb, selfW = p["basis"], p["comb"], p["self"]
    out = {t: _mm(hd[t], selfW) for t in hd}
    incoming = {}
    for r, (st, dt, src, dst) in enumerate(rels):
        incoming.setdefault(dt, []).append((r, st, src, dst))
    for dt, lst in incoming.items():
        acc = out[dt]
        for j, (r, st, src, dst) in enumerate(lst):
            Wr = comb[r, 0] * basis[0] + comb[r, 1] * basis[1]
            feat, deg = _sc_agg(hd[st], src, dst, hd[dt].shape[0])
            last = j == len(lst) - 1
            acc = _mm(feat, Wr, deg=deg, C=acc,
                      act="elu" if last else None)
        out[dt] = acc
    return out


def _mp_encode(h, eis, p):
    n = h.shape[0]
    zs = []
    ws = []
    for j, ei in enumerate(eis):
        feat, deg = _sc_agg(h, ei[0], ei[1], n)
        z = _mm(feat, p["W"][j], deg=deg, act="elu")
        zs.append(z)
        w = _att_logit(z, p["attW"], p["attb"], p["atta"])[0, 0] / n
        ws.append(w)
    beta = jax.nn.softmax(jnp.stack(ws))
    return _axpy(beta, zs[0], zs[1])


def _contrast_key(z_mp, z_sc, pk, p):
    a = _mm(z_mp, p["W1"], b=p["b1"], act="elu")
    zm = _mm(a, p["W2"], b=p["b2"], post="l2")
    a = _mm(z_sc, p["W1"], b=p["b1"], act="elu")
    zsn = _mm(a, p["W2"], b=p["b2"], post="l2")
    zspk = _sc_gather(zsn, pk)
    zmpk = _sc_gather(zm, pk)
    rs, cs, dg, pr, pc = _contrast_stats(zm, zsn, zspk, zmpk)
    return _contrast_loss(rs, cs, dg, pr, pc)[0, 0]


def kernel(x_drug, x_protein, x_sideeffect, x_disease, mp_drug_0, mp_drug_1,
           mp_protein_0, mp_protein_1, edge_dp, edge_ds, edge_pd, pos_drug,
           pos_protein, dti, params):
    p = params
    xs = {"drug": x_drug, "protein": x_protein,
          "sideeffect": x_sideeffect, "disease": x_disease}
    h = {t: _mm(xs[t], p["fc"][t]["W"], b=p["fc"][t]["b"], act="elu")
         for t in xs}

    rels = [("drug", "protein", edge_dp[0], edge_dp[1]),
            ("protein", "drug", edge_dp[1], edge_dp[0]),
            ("drug", "sideeffect", edge_ds[0], edge_ds[1]),
            ("sideeffect", "drug", edge_ds[1], edge_ds[0]),
            ("protein", "disease", edge_pd[0], edge_pd[1]),
            ("disease", "protein", edge_pd[1], edge_pd[0])]

    z_sc = _sc_layer(h, rels, p["sc"])
    z_sc = _sc_layer(z_sc, rels, p["sc2"])

    mps = {"drug": [mp_drug_0, mp_drug_1],
           "protein": [mp_protein_0, mp_protein_1]}
    z_mp = {k: _mp_encode(h[k], mps[k], p["mp"][k])
            for k in ("drug", "protein")}
    z_mp = {k: _mp_encode(z_mp[k], mps[k], p["mp2"][k])
            for k in ("drug", "protein")}

    loss = (_contrast_key(z_mp["drug"], z_sc["drug"], pos_drug,
                          p["contrast"])
            + _contrast_key(z_mp["protein"], z_sc["protein"], pos_protein,
                            p["contrast"])) / 2.0

    z_d = jnp.concatenate([z_sc["drug"], z_mp["drug"]], axis=1)
    z_p = jnp.concatenate([z_sc["protein"], z_mp["protein"]], axis=1)
    rows_d = _sc_gather(z_d, dti[:, 0])
    rows_p = _sc_gather(z_p, dti[:, 1])
    H = jnp.concatenate([rows_d, rows_p], axis=1)
    h1 = _mm(H, p["pred"]["W1"], b=p["pred"]["b1"], act="relu")
    W2 = jnp.pad(p["pred"]["W2"], ((0, 0), (0, OUT - 1)))
    b2 = jnp.pad(p["pred"]["b2"], (0, OUT - 1))
    o = _mm(h1, W2, b=b2, post="sigmoid")
    return loss, o[:, :1]


# speedup vs baseline: 5.5661x; 1.3693x over previous
import functools

import jax
import jax.numpy as jnp
from jax import lax
from jax.experimental import pallas as pl
from jax.experimental.pallas import tpu as pltpu
from jax.experimental.pallas import tpu_sc as plsc

F32 = jnp.float32
OUT = 128
TAU = 0.8
LAM = 0.5
NC = 2
NS = 16
NW = NC * NS
CH = 128


def _cdiv(a, b):
    return (a + b - 1) // b


def _elu(x):
    return jnp.where(x > 0, x, jnp.exp(jnp.minimum(x, 0.0)) - 1.0)


KB = 2


@functools.lru_cache(maxsize=None)
def _sc_agg_call(n_src, n_dst, e_pad):
    n_pad = _cdiv(n_dst, NS * 8) * NS * 8
    rps = n_pad // NS
    s_pad = _cdiv(n_src, NS * 8) * NS * 8
    srps = s_pad // NS
    tm = e_pad // (KB * CH)
    mesh = plsc.VectorSubcoreMesh(core_axis_name="c", subcore_axis_name="s")

    @functools.partial(
        pl.kernel,
        mesh=mesh,
        out_type=(
            jax.ShapeDtypeStruct((NC, n_pad, OUT), F32),
            jax.ShapeDtypeStruct((NC * n_pad,), F32),
        ),
        scratch_types=[
            pltpu.VMEM_SHARED((s_pad, OUT), F32),
            pltpu.VMEM_SHARED((n_pad, OUT), F32),
            pltpu.VMEM_SHARED((n_pad,), F32),
            pltpu.VMEM((KB * CH,), jnp.int32),
            pltpu.VMEM((KB, CH), jnp.int32),
            pltpu.VMEM((KB, CH, OUT), F32),
            pltpu.VMEM((CH,), F32),
            pltpu.VMEM((rps,), F32),
            pltpu.SemaphoreType.DMA,
        ],
    )
    def k(table, src, dst, zf, zd, ones, o_feat, o_deg,
          tab_sh, feat_sh, deg_sh, sidx, didx, rows, ones_v, dbuf, sem):
        cid = lax.axis_index("c")
        sid = lax.axis_index("s")
        wid = sid * NC + cid
        pltpu.sync_copy(table.at[pl.ds(sid * srps, srps)],
                        tab_sh.at[pl.ds(sid * srps, srps)])
        pltpu.sync_copy(zf.at[pl.ds(sid * rps, rps)],
                        feat_sh.at[pl.ds(sid * rps, rps)])
        pltpu.sync_copy(zd.at[pl.ds(sid * rps, rps)], dbuf)
        pltpu.sync_copy(dbuf, deg_sh.at[pl.ds(sid * rps, rps)])
        pltpu.sync_copy(ones, ones_v)
        plsc.subcore_barrier()

        nj = (tm - 1 - wid) // NW + 1

        @pl.loop(0, nj)
        def _(j):
            base = (wid + j * NW) * KB * CH
            cps = [pltpu.async_copy(src.at[pl.ds(base, KB * CH)], sidx, sem)]
            for b in range(KB):
                cps.append(pltpu.async_copy(
                    dst.at[pl.ds(base + b * CH, CH)], didx.at[b], sem))
            for c in cps:
                c.wait()
            cps = [pltpu.async_copy(
                tab_sh.at[sidx.at[pl.ds(b * CH, CH)]], rows.at[b], sem)
                for b in range(KB)]
            for c in cps:
                c.wait()
            cps = []
            for b in range(KB):
                cps.append(pltpu.async_copy(
                    rows.at[b], feat_sh.at[didx.at[b]], sem, add=True))
                cps.append(pltpu.async_copy(
                    ones_v, deg_sh.at[didx.at[b]], sem, add=True))
            for c in cps:
                c.wait()

        plsc.subcore_barrier()
        pltpu.sync_copy(feat_sh.at[pl.ds(sid * rps, rps)],
                        o_feat.at[cid, pl.ds(sid * rps, rps)])
        pltpu.sync_copy(deg_sh.at[pl.ds(sid * rps, rps)], dbuf)
        pltpu.sync_copy(dbuf, o_deg.at[pl.ds(cid * n_pad + sid * rps, rps)])

    return k, n_pad, s_pad


def _sc_agg(table, src, dst, n_dst):
    n_src, _ = table.shape
    e = src.shape[0]
    e_pad = _cdiv(e, KB * CH) * KB * CH
    k, n_pad, s_pad = _sc_agg_call(n_src, n_dst, e_pad)
    tab = jnp.pad(table, ((0, s_pad - n_src), (0, 0)))
    src = src.astype(jnp.int32)
    dst = dst.astype(jnp.int32)
    if e_pad != e:
        src = jnp.concatenate([src, jnp.zeros((e_pad - e,), jnp.int32)])
        dst = jnp.concatenate(
            [dst, jnp.full((e_pad - e,), min(n_dst, n_pad - 1), jnp.int32)])
    zf = jnp.zeros((n_pad, OUT), F32)
    zd = jnp.zeros((n_pad,), F32)
    ones = jnp.ones((CH,), F32)
    feat, deg = k(tab, src, dst, zf, zd, ones)
    return feat[:, :n_dst, :], deg.reshape(NC, n_pad)[:, :n_dst]


@functools.lru_cache(maxsize=None)
def _sc_gather_call(n_src, d, m):
    t_chunks = m // CH
    mesh = plsc.VectorSubcoreMesh(core_axis_name="c", subcore_axis_name="s")

    @functools.partial(
        pl.kernel,
        mesh=mesh,
        out_type=jax.ShapeDtypeStruct((m, d), F32),
        scratch_types=[
            pltpu.VMEM((CH,), jnp.int32),
            pltpu.VMEM((CH, d), F32),
        ],
    )
    def k(table, idx, out, idx_v, rows):
        cid = lax.axis_index("c")
        sid = lax.axis_index("s")
        wid = sid * NC + cid
        nj = (t_chunks - 1 - wid) // NW + 1

        @pl.loop(0, nj)
        def _(j):
            base = (wid + j * NW) * CH
            pltpu.sync_copy(idx.at[pl.ds(base, CH)], idx_v)
            pltpu.sync_copy(table.at[idx_v], rows)
            pltpu.sync_copy(rows, out.at[pl.ds(base, CH)])

    return k


def _sc_gather(table, idx):
    n_src, d = table.shape
    m = idx.shape[0]
    m_pad = _cdiv(m, CH) * CH
    idx = idx.astype(jnp.int32)
    if m_pad != m:
        idx = jnp.concatenate([idx, jnp.zeros((m_pad - m,), jnp.int32)])
    k = _sc_gather_call(n_src, d, m_pad)
    return k(table, idx)[:m]


def _apply_act(y, act):
    if act == "elu":
        return _elu(y)
    if act == "relu":
        return jnp.maximum(y, 0.0)
    return y


def _apply_post(y, post):
    if post == "l2":
        nrm = jnp.sqrt(jnp.sum(y * y, axis=1, keepdims=True))
        return y / (nrm + 1e-8)
    if post == "sigmoid":
        return 1.0 / (1.0 + jnp.exp(-y))
    return y


def _mm(x, W, *, deg=None, C=None, b=None, act=None, post=None, bm=512):
    parts = x.ndim == 3
    n = x.shape[1] if parts else x.shape[0]
    K = x.shape[-1]
    M = W.shape[1]
    grid = _cdiv(n, bm)
    in_specs = []
    args = []
    if parts:
        in_specs.append(pl.BlockSpec((2, bm, K), lambda i: (0, i, 0)))
    else:
        in_specs.append(pl.BlockSpec((bm, K), lambda i: (i, 0)))
    args.append(x)
    has_deg = deg is not None
    if has_deg:
        in_specs.append(pl.BlockSpec((2, bm), lambda i: (0, i)))
        args.append(deg)
    in_specs.append(pl.BlockSpec((K, M), lambda i: (0, 0)))
    args.append(W)
    has_b = b is not None
    if has_b:
        in_specs.append(pl.BlockSpec((M,), lambda i: (0,)))
        args.append(b)
    has_c = C is not None
    if has_c:
        in_specs.append(pl.BlockSpec((bm, M), lambda i: (i, 0)))
        args.append(C)

    def body(*refs):
        it = iter(refs)
        x_ref = next(it)
        d_ref = next(it) if has_deg else None
        w_ref = next(it)
        b_ref = next(it) if has_b else None
        c_ref = next(it) if has_c else None
        o_ref = next(it)
        if parts:
            xv = x_ref[0] + x_ref[1]
        else:
            xv = x_ref[...]
        if has_deg:
            dsum = d_ref[0] + d_ref[1]
            scale = 1.0 / jnp.maximum(dsum, 1.0)
            xv = xv * scale[:, None]
        y = jnp.dot(xv, w_ref[...], preferred_element_type=F32)
        if has_b:
            y = y + b_ref[...][None, :]
        if has_c:
            y = y + c_ref[...]
        y = _apply_act(y, act)
        y = _apply_post(y, post)
        o_ref[...] = y

    return pl.pallas_call(
        body,
        grid=(grid,),
        in_specs=in_specs,
        out_specs=pl.BlockSpec((bm, M), lambda i: (i, 0)),
        out_shape=jax.ShapeDtypeStruct((n, M), F32),
    )(*args)


def _att_logit(z, attW, attb, atta, bm=512):
    n = z.shape[0]
    grid = _cdiv(n, bm)

    def body(z_ref, w_ref, b_ref, a_ref, o_ref):
        i = pl.program_id(0)
        t = jnp.tanh(jnp.dot(z_ref[...], w_ref[...],
                             preferred_element_type=F32) + b_ref[...][None, :])
        d = jnp.sum(t * a_ref[...][None, :], axis=1, keepdims=True)
        rid = lax.broadcasted_iota(jnp.int32, (bm, 1), 0) + i * bm
        d = jnp.where(rid < n, d, 0.0)
        s = jnp.sum(d)

        @pl.when(i == 0)
        def _():
            o_ref[0, 0] = s

        @pl.when(i > 0)
        def _():
            o_ref[0, 0] = o_ref[0, 0] + s

    return pl.pallas_call(
        body,
        grid=(grid,),
        in_specs=[
            pl.BlockSpec((bm, OUT), lambda i: (i, 0)),
            pl.BlockSpec((OUT, OUT), lambda i: (0, 0)),
            pl.BlockSpec((OUT,), lambda i: (0,)),
            pl.BlockSpec((OUT,), lambda i: (0,)),
        ],
        out_specs=pl.BlockSpec(memory_space=pltpu.SMEM),
        out_shape=jax.ShapeDtypeStruct((1, 1), F32),
    )(z, attW, attb, atta)


def _axpy(ab, x, y, bm=512):
    n = x.shape[0]
    grid = _cdiv(n, bm)

    def body(ab_ref, x_ref, y_ref, o_ref):
        o_ref[...] = ab_ref[0] * x_ref[...] + ab_ref[1] * y_ref[...]

    return pl.pallas_call(
        body,
        grid=(grid,),
        in_specs=[
            pl.BlockSpec(memory_space=pltpu.SMEM),
            pl.BlockSpec((bm, OUT), lambda i: (i, 0)),
            pl.BlockSpec((bm, OUT), lambda i: (i, 0)),
        ],
        out_specs=pl.BlockSpec((bm, OUT), lambda i: (i, 0)),
        out_shape=jax.ShapeDtypeStruct((n, OUT), F32),
    )(ab, x, y)


def _contrast_stats(zm, zs, zspk, zmpk, bm=512):
    n = zm.shape[0]
    grid = _cdiv(n, bm)

    def body(zm_ref, zsf_ref, zsb_ref, zspk_ref, zmpk_ref,
             rs_ref, cs_ref, dg_ref, pr_ref, pc_ref):
        i = pl.program_id(0)
        zmb = zm_ref[...]
        zsf = zsf_ref[...]
        logits = lax.dot_general(zmb, zsf, (((1,), (1,)), ((), ())),
                                 preferred_element_type=F32) * (1.0 / TAU)
        e = jnp.exp(logits)
        rid = lax.broadcasted_iota(jnp.int32, (bm, 1), 0) + i * bm
        valid = rid < n
        rs_ref[...] = jnp.sum(e, axis=1)
        cs = jnp.sum(jnp.where(valid, e, 0.0), axis=0)

        @pl.when(i == 0)
        def _():
            cs_ref[...] = cs

        @pl.when(i > 0)
        def _():
            cs_ref[...] = cs_ref[...] + cs

        zsb = zsb_ref[...]
        dg_ref[...] = jnp.exp(jnp.sum(zmb * zsb, axis=1) * (1.0 / TAU))
        pr_ref[...] = jnp.exp(jnp.sum(zmb * zspk_ref[...], axis=1) * (1.0 / TAU))
        pc_ref[...] = jnp.exp(jnp.sum(zmpk_ref[...] * zsb, axis=1) * (1.0 / TAU))

    sd = jax.ShapeDtypeStruct((n,), F32)
    return pl.pallas_call(
        body,
        grid=(grid,),
        in_specs=[
            pl.BlockSpec((bm, OUT), lambda i: (i, 0)),
            pl.BlockSpec((n, OUT), lambda i: (0, 0)),
            pl.BlockSpec((bm, OUT), lambda i: (i, 0)),
            pl.BlockSpec((bm, OUT), lambda i: (i, 0)),
            pl.BlockSpec((bm, OUT), lambda i: (i, 0)),
        ],
        out_specs=[
            pl.BlockSpec((bm,), lambda i: (i,)),
            pl.BlockSpec((n,), lambda i: (0,)),
            pl.BlockSpec((bm,), lambda i: (i,)),
            pl.BlockSpec((bm,), lambda i: (i,)),
            pl.BlockSpec((bm,), lambda i: (i,)),
        ],
        out_shape=(sd, sd, sd, sd, sd),
    )(zm, zs, zs, zspk, zmpk)


def _contrast_loss(rs, cs, dg, pr, pc):
    n = rs.shape[0]

    def body(rs_ref, cs_ref, dg_ref, pr_ref, pc_ref, o_ref):
        l_mp = -jnp.sum(jnp.log((pr_ref[...] + dg_ref[...]) /
                                (rs_ref[...] + 1e-8) + 1e-8)) / n
        l_sc = -jnp.sum(jnp.log((pc_ref[...] + dg_ref[...]) /
                                (cs_ref[...] + 1e-8) + 1e-8)) / n
        o_ref[0, 0] = LAM * l_mp + (1.0 - LAM) * l_sc

    return pl.pallas_call(
        body,
        in_specs=[pl.BlockSpec((n,), lambda: (0,))] * 5,
        out_specs=pl.BlockSpec(memory_space=pltpu.SMEM),
        out_shape=jax.ShapeDtypeStruct((1, 1), F32),
    )(rs, cs, dg, pr, pc)


def _sc_layer(hd, rels, p):
    basis, comb, selfW = p["basis"], p["comb"], p["self"]
    out = {t: _mm(hd[t], selfW) for t in hd}
    incoming = {}
    for r, (st, dt, src, dst) in enumerate(rels):
        incoming.setdefault(dt, []).append((r, st, src, dst))
    for dt, lst in incoming.items():
        acc = out[dt]
        for j, (r, st, src, dst) in enumerate(lst):
            Wr = comb[r, 0] * basis[0] + comb[r, 1] * basis[1]
            feat, deg = _sc_agg(hd[st], src, dst, hd[dt].shape[0])
            last = j == len(lst) - 1
            acc = _mm(feat, Wr, deg=deg, C=acc,
                      act="elu" if last else None)
        out[dt] = acc
    return out


def _mp_encode(h, eis, p):
    n = h.shape[0]
    zs = []
    ws = []
    for j, ei in enumerate(eis):
        feat, deg = _sc_agg(h, ei[0], ei[1], n)
        z = _mm(feat, p["W"][j], deg=deg, act="elu")
        zs.append(z)
        w = _att_logit(z, p["attW"], p["attb"], p["atta"])[0, 0] / n
        ws.append(w)
    beta = jax.nn.softmax(jnp.stack(ws))
    return _axpy(beta, zs[0], zs[1])


def _contrast_key(z_mp, z_sc, pk, p):
    a = _mm(z_mp, p["W1"], b=p["b1"], act="elu")
    zm = _mm(a, p["W2"], b=p["b2"], post="l2")
    a = _mm(z_sc, p["W1"], b=p["b1"], act="elu")
    zsn = _mm(a, p["W2"], b=p["b2"], post="l2")
    zspk = _sc_gather(zsn, pk)
    zmpk = _sc_gather(zm, pk)
    rs, cs, dg, pr, pc = _contrast_stats(zm, zsn, zspk, zmpk)
    return _contrast_loss(rs, cs, dg, pr, pc)[0, 0]


def kernel(x_drug, x_protein, x_sideeffect, x_disease, mp_drug_0, mp_drug_1,
           mp_protein_0, mp_protein_1, edge_dp, edge_ds, edge_pd, pos_drug,
           pos_protein, dti, params):
    p = params
    xs = {"drug": x_drug, "protein": x_protein,
          "sideeffect": x_sideeffect, "disease": x_disease}
    h = {t: _mm(xs[t], p["fc"][t]["W"], b=p["fc"][t]["b"], act="elu")
         for t in xs}

    rels = [("drug", "protein", edge_dp[0], edge_dp[1]),
            ("protein", "drug", edge_dp[1], edge_dp[0]),
            ("drug", "sideeffect", edge_ds[0], edge_ds[1]),
            ("sideeffect", "drug", edge_ds[1], edge_ds[0]),
            ("protein", "disease", edge_pd[0], edge_pd[1]),
            ("disease", "protein", edge_pd[1], edge_pd[0])]

    z_sc = _sc_layer(h, rels, p["sc"])
    z_sc = _sc_layer(z_sc, rels, p["sc2"])

    mps = {"drug": [mp_drug_0, mp_drug_1],
           "protein": [mp_protein_0, mp_protein_1]}
    z_mp = {k: _mp_encode(h[k], mps[k], p["mp"][k])
            for k in ("drug", "protein")}
    z_mp = {k: _mp_encode(z_mp[k], mps[k], p["mp2"][k])
            for k in ("drug", "protein")}

    loss = (_contrast_key(z_mp["drug"], z_sc["drug"], pos_drug,
                          p["contrast"])
            + _contrast_key(z_mp["protein"], z_sc["protein"], pos_protein,
                            p["contrast"])) / 2.0

    z_d = jnp.concatenate([z_sc["drug"], z_mp["drug"]], axis=1)
    z_p = jnp.concatenate([z_sc["protein"], z_mp["protein"]], axis=1)
    rows_d = _sc_gather(z_d, dti[:, 0])
    rows_p = _sc_gather(z_p, dti[:, 1])
    H = jnp.concatenate([rows_d, rows_p], axis=1)
    h1 = _mm(H, p["pred"]["W1"], b=p["pred"]["b1"], act="relu")
    W2 = jnp.pad(p["pred"]["W2"], ((0, 0), (0, OUT - 1)))
    b2 = jnp.pad(p["pred"]["b2"], (0, OUT - 1))
    o = _mm(h1, W2, b=b2, post="sigmoid")
    return loss, o[:, :1]


# speedup vs baseline: 5.6256x; 1.0107x over previous
import functools

import jax
import jax.numpy as jnp
from jax import lax
from jax.experimental import pallas as pl
from jax.experimental.pallas import tpu as pltpu
from jax.experimental.pallas import tpu_sc as plsc

F32 = jnp.float32
OUT = 128
TAU = 0.8
LAM = 0.5
NC = 2
NS = 16
NW = NC * NS
CH = 128


def _cdiv(a, b):
    return (a + b - 1) // b


def _elu(x):
    return jnp.where(x > 0, x, jnp.exp(jnp.minimum(x, 0.0)) - 1.0)


KB = 2


@functools.lru_cache(maxsize=None)
def _sc_agg_call(n_src, n_dst, e_pad):
    n_pad = _cdiv(n_dst, NS * 8) * NS * 8
    rps = n_pad // NS
    s_pad = _cdiv(n_src, NS * 8) * NS * 8
    srps = s_pad // NS
    tm = e_pad // (KB * CH)
    mesh = plsc.VectorSubcoreMesh(core_axis_name="c", subcore_axis_name="s")

    @functools.partial(
        pl.kernel,
        mesh=mesh,
        out_type=(
            jax.ShapeDtypeStruct((NC, n_pad, OUT), F32),
            jax.ShapeDtypeStruct((NC * n_pad,), F32),
        ),
        scratch_types=[
            pltpu.VMEM_SHARED((s_pad, OUT), F32),
            pltpu.VMEM_SHARED((n_pad, OUT), F32),
            pltpu.VMEM_SHARED((n_pad,), F32),
            pltpu.VMEM((2, KB * CH), jnp.int32),
            pltpu.VMEM((2, KB, CH), jnp.int32),
            pltpu.VMEM((KB, CH, OUT), F32),
            pltpu.VMEM((CH,), F32),
            pltpu.VMEM((rps,), F32),
            pltpu.SemaphoreType.DMA,
            pltpu.SemaphoreType.DMA,
            pltpu.SemaphoreType.DMA,
        ],
    )
    def k(table, src, dst, zf, zd, ones, o_feat, o_deg,
          tab_sh, feat_sh, deg_sh, sidx, didx, rows, ones_v, dbuf,
          sem_i, sem_g, sem_s):
        cid = lax.axis_index("c")
        sid = lax.axis_index("s")
        wid = sid * NC + cid

        def fire_idx(j, p):
            base = (wid + j * NW) * KB * CH
            pltpu.async_copy(src.at[pl.ds(base, KB * CH)], sidx.at[p], sem_i)
            for b in range(KB):
                pltpu.async_copy(dst.at[pl.ds(base + b * CH, CH)],
                                 didx.at[p, b], sem_i)

        def drain_idx(p):
            pltpu.make_async_copy(src.at[pl.ds(0, KB * CH)], sidx.at[p],
                                  sem_i).wait()
            for b in range(KB):
                pltpu.make_async_copy(dst.at[pl.ds(0, CH)], didx.at[p, b],
                                      sem_i).wait()

        pltpu.sync_copy(table.at[pl.ds(sid * srps, srps)],
                        tab_sh.at[pl.ds(sid * srps, srps)])
        pltpu.sync_copy(zf.at[pl.ds(sid * rps, rps)],
                        feat_sh.at[pl.ds(sid * rps, rps)])
        pltpu.sync_copy(zd.at[pl.ds(sid * rps, rps)], dbuf)
        pltpu.sync_copy(dbuf, deg_sh.at[pl.ds(sid * rps, rps)])
        pltpu.sync_copy(ones, ones_v)
        plsc.subcore_barrier()

        nj = (tm - 1 - wid) // NW + 1
        fire_idx(0, 0)
        fire_idx(1, 1)

        @pl.loop(0, (nj + 1) // 2)
        def _(h):
            for p in range(2):
                jj = 2 * h + p

                @pl.when(jj < nj)
                def _():
                    drain_idx(p)
                    cps = [pltpu.async_copy(
                        tab_sh.at[sidx.at[p, pl.ds(b * CH, CH)]],
                        rows.at[b], sem_g) for b in range(KB)]
                    for c in cps:
                        c.wait()
                    cps = []
                    for b in range(KB):
                        cps.append(pltpu.async_copy(
                            rows.at[b], feat_sh.at[didx.at[p, b]],
                            sem_s, add=True))
                        cps.append(pltpu.async_copy(
                            ones_v, deg_sh.at[didx.at[p, b]],
                            sem_s, add=True))
                    for c in cps:
                        c.wait()

                    @pl.when(jj + 2 < nj)
                    def _():
                        fire_idx(jj + 2, p)

        plsc.subcore_barrier()
        pltpu.sync_copy(feat_sh.at[pl.ds(sid * rps, rps)],
                        o_feat.at[cid, pl.ds(sid * rps, rps)])
        pltpu.sync_copy(deg_sh.at[pl.ds(sid * rps, rps)], dbuf)
        pltpu.sync_copy(dbuf, o_deg.at[pl.ds(cid * n_pad + sid * rps, rps)])

    return k, n_pad, s_pad


def _sc_agg(table, src, dst, n_dst):
    n_src, _ = table.shape
    e = src.shape[0]
    e_pad = _cdiv(e, KB * CH) * KB * CH
    k, n_pad, s_pad = _sc_agg_call(n_src, n_dst, e_pad)
    tab = jnp.pad(table, ((0, s_pad - n_src), (0, 0)))
    src = src.astype(jnp.int32)
    dst = dst.astype(jnp.int32)
    if e_pad != e:
        src = jnp.concatenate([src, jnp.zeros((e_pad - e,), jnp.int32)])
        dst = jnp.concatenate(
            [dst, jnp.full((e_pad - e,), min(n_dst, n_pad - 1), jnp.int32)])
    zf = jnp.zeros((n_pad, OUT), F32)
    zd = jnp.zeros((n_pad,), F32)
    ones = jnp.ones((CH,), F32)
    feat, deg = k(tab, src, dst, zf, zd, ones)
    return feat[:, :n_dst, :], deg.reshape(NC, n_pad)[:, :n_dst]


@functools.lru_cache(maxsize=None)
def _sc_gather_call(n_src, d, m):
    t_chunks = m // CH
    mesh = plsc.VectorSubcoreMesh(core_axis_name="c", subcore_axis_name="s")

    @functools.partial(
        pl.kernel,
        mesh=mesh,
        out_type=jax.ShapeDtypeStruct((m, d), F32),
        scratch_types=[
            pltpu.VMEM((CH,), jnp.int32),
            pltpu.VMEM((CH, d), F32),
        ],
    )
    def k(table, idx, out, idx_v, rows):
        cid = lax.axis_index("c")
        sid = lax.axis_index("s")
        wid = sid * NC + cid
        nj = (t_chunks - 1 - wid) // NW + 1

        @pl.loop(0, nj)
        def _(j):
            base = (wid + j * NW) * CH
            pltpu.sync_copy(idx.at[pl.ds(base, CH)], idx_v)
            pltpu.sync_copy(table.at[idx_v], rows)
            pltpu.sync_copy(rows, out.at[pl.ds(base, CH)])

    return k


def _sc_gather(table, idx):
    n_src, d = table.shape
    m = idx.shape[0]
    m_pad = _cdiv(m, CH) * CH
    idx = idx.astype(jnp.int32)
    if m_pad != m:
        idx = jnp.concatenate([idx, jnp.zeros((m_pad - m,), jnp.int32)])
    k = _sc_gather_call(n_src, d, m_pad)
    return k(table, idx)[:m]


def _apply_act(y, act):
    if act == "elu":
        return _elu(y)
    if act == "relu":
        return jnp.maximum(y, 0.0)
    return y


def _apply_post(y, post):
    if post == "l2":
        nrm = jnp.sqrt(jnp.sum(y * y, axis=1, keepdims=True))
        return y / (nrm + 1e-8)
    if post == "sigmoid":
        return 1.0 / (1.0 + jnp.exp(-y))
    return y


def _mm(x, W, *, deg=None, C=None, b=None, act=None, post=None, bm=512):
    parts = x.ndim == 3
    n = x.shape[1] if parts else x.shape[0]
    K = x.shape[-1]
    M = W.shape[1]
    grid = _cdiv(n, bm)
    in_specs = []
    args = []
    if parts:
        in_specs.append(pl.BlockSpec((2, bm, K), lambda i: (0, i, 0)))
    else:
        in_specs.append(pl.BlockSpec((bm, K), lambda i: (i, 0)))
    args.append(x)
    has_deg = deg is not None
    if has_deg:
        in_specs.append(pl.BlockSpec((2, bm), lambda i: (0, i)))
        args.append(deg)
    in_specs.append(pl.BlockSpec((K, M), lambda i: (0, 0)))
    args.append(W)
    has_b = b is not None
    if has_b:
        in_specs.append(pl.BlockSpec((M,), lambda i: (0,)))
        args.append(b)
    has_c = C is not None
    if has_c:
        in_specs.append(pl.BlockSpec((bm, M), lambda i: (i, 0)))
        args.append(C)

    def body(*refs):
        it = iter(refs)
        x_ref = next(it)
        d_ref = next(it) if has_deg else None
        w_ref = next(it)
        b_ref = next(it) if has_b else None
        c_ref = next(it) if has_c else None
        o_ref = next(it)
        if parts:
            xv = x_ref[0] + x_ref[1]
        else:
            xv = x_ref[...]
        if has_deg:
            dsum = d_ref[0] + d_ref[1]
            scale = 1.0 / jnp.maximum(dsum, 1.0)
            xv = xv * scale[:, None]
        y = jnp.dot(xv, w_ref[...], preferred_element_type=F32)
        if has_b:
            y = y + b_ref[...][None, :]
        if has_c:
            y = y + c_ref[...]
        y = _apply_act(y, act)
        y = _apply_post(y, post)
        o_ref[...] = y

    return pl.pallas_call(
        body,
        grid=(grid,),
        in_specs=in_specs,
        out_specs=pl.BlockSpec((bm, M), lambda i: (i, 0)),
        out_shape=jax.ShapeDtypeStruct((n, M), F32),
    )(*args)


def _att_logit(z, attW, attb, atta, bm=512):
    n = z.shape[0]
    grid = _cdiv(n, bm)

    def body(z_ref, w_ref, b_ref, a_ref, o_ref):
        i = pl.program_id(0)
        t = jnp.tanh(jnp.dot(z_ref[...], w_ref[...],
                             preferred_element_type=F32) + b_ref[...][None, :])
        d = jnp.sum(t * a_ref[...][None, :], axis=1, keepdims=True)
        rid = lax.broadcasted_iota(jnp.int32, (bm, 1), 0) + i * bm
        d = jnp.where(rid < n, d, 0.0)
        s = jnp.sum(d)

        @pl.when(i == 0)
        def _():
            o_ref[0, 0] = s

        @pl.when(i > 0)
        def _():
            o_ref[0, 0] = o_ref[0, 0] + s

    return pl.pallas_call(
        body,
        grid=(grid,),
        in_specs=[
            pl.BlockSpec((bm, OUT), lambda i: (i, 0)),
            pl.BlockSpec((OUT, OUT), lambda i: (0, 0)),
            pl.BlockSpec((OUT,), lambda i: (0,)),
            pl.BlockSpec((OUT,), lambda i: (0,)),
        ],
        out_specs=pl.BlockSpec(memory_space=pltpu.SMEM),
        out_shape=jax.ShapeDtypeStruct((1, 1), F32),
    )(z, attW, attb, atta)


def _axpy(ab, x, y, bm=512):
    n = x.shape[0]
    grid = _cdiv(n, bm)

    def body(ab_ref, x_ref, y_ref, o_ref):
        o_ref[...] = ab_ref[0] * x_ref[...] + ab_ref[1] * y_ref[...]

    return pl.pallas_call(
        body,
        grid=(grid,),
        in_specs=[
            pl.BlockSpec(memory_space=pltpu.SMEM),
            pl.BlockSpec((bm, OUT), lambda i: (i, 0)),
            pl.BlockSpec((bm, OUT), lambda i: (i, 0)),
        ],
        out_specs=pl.BlockSpec((bm, OUT), lambda i: (i, 0)),
        out_shape=jax.ShapeDtypeStruct((n, OUT), F32),
    )(ab, x, y)


def _contrast_stats(zm, zs, zspk, zmpk, bm=512):
    n = zm.shape[0]
    grid = _cdiv(n, bm)

    def body(zm_ref, zsf_ref, zsb_ref, zspk_ref, zmpk_ref,
             rs_ref, cs_ref, dg_ref, pr_ref, pc_ref):
        i = pl.program_id(0)
        zmb = zm_ref[...]
        zsf = zsf_ref[...]
        logits = lax.dot_general(zmb, zsf, (((1,), (1,)), ((), ())),
                                 preferred_element_type=F32) * (1.0 / TAU)
        e = jnp.exp(logits)
        rid = lax.broadcasted_iota(jnp.int32, (bm, 1), 0) + i * bm
        valid = rid < n
        rs_ref[...] = jnp.sum(e, axis=1)
        cs = jnp.sum(jnp.where(valid, e, 0.0), axis=0)

        @pl.when(i == 0)
        def _():
            cs_ref[...] = cs

        @pl.when(i > 0)
        def _():
            cs_ref[...] = cs_ref[...] + cs

        zsb = zsb_ref[...]
        dg_ref[...] = jnp.exp(jnp.sum(zmb * zsb, axis=1) * (1.0 / TAU))
        pr_ref[...] = jnp.exp(jnp.sum(zmb * zspk_ref[...], axis=1) * (1.0 / TAU))
        pc_ref[...] = jnp.exp(jnp.sum(zmpk_ref[...] * zsb, axis=1) * (1.0 / TAU))

    sd = jax.ShapeDtypeStruct((n,), F32)
    return pl.pallas_call(
        body,
        grid=(grid,),
        in_specs=[
            pl.BlockSpec((bm, OUT), lambda i: (i, 0)),
            pl.BlockSpec((n, OUT), lambda i: (0, 0)),
            pl.BlockSpec((bm, OUT), lambda i: (i, 0)),
            pl.BlockSpec((bm, OUT), lambda i: (i, 0)),
            pl.BlockSpec((bm, OUT), lambda i: (i, 0)),
        ],
        out_specs=[
            pl.BlockSpec((bm,), lambda i: (i,)),
            pl.BlockSpec((n,), lambda i: (0,)),
            pl.BlockSpec((bm,), lambda i: (i,)),
            pl.BlockSpec((bm,), lambda i: (i,)),
            pl.BlockSpec((bm,), lambda i: (i,)),
        ],
        out_shape=(sd, sd, sd, sd, sd),
    )(zm, zs, zs, zspk, zmpk)


def _contrast_loss(rs, cs, dg, pr, pc):
    n = rs.shape[0]

    def body(rs_ref, cs_ref, dg_ref, pr_ref, pc_ref, o_ref):
        l_mp = -jnp.sum(jnp.log((pr_ref[...] + dg_ref[...]) /
                                (rs_ref[...] + 1e-8) + 1e-8)) / n
        l_sc = -jnp.sum(jnp.log((pc_ref[...] + dg_ref[...]) /
                                (cs_ref[...] + 1e-8) + 1e-8)) / n
        o_ref[0, 0] = LAM * l_mp + (1.0 - LAM) * l_sc

    return pl.pallas_call(
        body,
        in_specs=[pl.BlockSpec((n,), lambda: (0,))] * 5,
        out_specs=pl.BlockSpec(memory_space=pltpu.SMEM),
        out_shape=jax.ShapeDtypeStruct((1, 1), F32),
    )(rs, cs, dg, pr, pc)


def _sc_layer(hd, rels, p):
    basis, comb, selfW = p["basis"], p["comb"], p["self"]
    out = {t: _mm(hd[t], selfW) for t in hd}
    incoming = {}
    for r, (st, dt, src, dst) in enumerate(rels):
        incoming.setdefault(dt, []).append((r, st, src, dst))
    for dt, lst in incoming.items():
        acc = out[dt]
        for j, (r, st, src, dst) in enumerate(lst):
            Wr = comb[r, 0] * basis[0] + comb[r, 1] * basis[1]
            feat, deg = _sc_agg(hd[st], src, dst, hd[dt].shape[0])
            last = j == len(lst) - 1
            acc = _mm(feat, Wr, deg=deg, C=acc,
                      act="elu" if last else None)
        out[dt] = acc
    return out


def _mp_encode(h, eis, p):
    n = h.shape[0]
    zs = []
    ws = []
    for j, ei in enumerate(eis):
        feat, deg = _sc_agg(h, ei[0], ei[1], n)
        z = _mm(feat, p["W"][j], deg=deg, act="elu")
        zs.append(z)
        w = _att_logit(z, p["attW"], p["attb"], p["atta"])[0, 0] / n
        ws.append(w)
    beta = jax.nn.softmax(jnp.stack(ws))
    return _axpy(beta, zs[0], zs[1])


def _contrast_key(z_mp, z_sc, pk, p):
    a = _mm(z_mp, p["W1"], b=p["b1"], act="elu")
    zm = _mm(a, p["W2"], b=p["b2"], post="l2")
    a = _mm(z_sc, p["W1"], b=p["b1"], act="elu")
    zsn = _mm(a, p["W2"], b=p["b2"], post="l2")
    zspk = _sc_gather(zsn, pk)
    zmpk = _sc_gather(zm, pk)
    rs, cs, dg, pr, pc = _contrast_stats(zm, zsn, zspk, zmpk)
    return _contrast_loss(rs, cs, dg, pr, pc)[0, 0]


def kernel(x_drug, x_protein, x_sideeffect, x_disease, mp_drug_0, mp_drug_1,
           mp_protein_0, mp_protein_1, edge_dp, edge_ds, edge_pd, pos_drug,
           pos_protein, dti, params):
    p = params
    xs = {"drug": x_drug, "protein": x_protein,
          "sideeffect": x_sideeffect, "disease": x_disease}
    h = {t: _mm(xs[t], p["fc"][t]["W"], b=p["fc"][t]["b"], act="elu")
         for t in xs}

    rels = [("drug", "protein", edge_dp[0], edge_dp[1]),
            ("protein", "drug", edge_dp[1], edge_dp[0]),
            ("drug", "sideeffect", edge_ds[0], edge_ds[1]),
            ("sideeffect", "drug", edge_ds[1], edge_ds[0]),
            ("protein", "disease", edge_pd[0], edge_pd[1]),
            ("disease", "protein", edge_pd[1], edge_pd[0])]

    z_sc = _sc_layer(h, rels, p["sc"])
    z_sc = _sc_layer(z_sc, rels, p["sc2"])

    mps = {"drug": [mp_drug_0, mp_drug_1],
           "protein": [mp_protein_0, mp_protein_1]}
    z_mp = {k: _mp_encode(h[k], mps[k], p["mp"][k])
            for k in ("drug", "protein")}
    z_mp = {k: _mp_encode(z_mp[k], mps[k], p["mp2"][k])
            for k in ("drug", "protein")}

    loss = (_contrast_key(z_mp["drug"], z_sc["drug"], pos_drug,
                          p["contrast"])
            + _contrast_key(z_mp["protein"], z_sc["protein"], pos_protein,
                            p["contrast"])) / 2.0

    z_d = jnp.concatenate([z_sc["drug"], z_mp["drug"]], axis=1)
    z_p = jnp.concatenate([z_sc["protein"], z_mp["protein"]], axis=1)
    rows_d = _sc_gather(z_d, dti[:, 0])
    rows_p = _sc_gather(z_p, dti[:, 1])
    H = jnp.concatenate([rows_d, rows_p], axis=1)
    h1 = _mm(H, p["pred"]["W1"], b=p["pred"]["b1"], act="relu")
    W2 = jnp.pad(p["pred"]["W2"], ((0, 0), (0, OUT - 1)))
    b2 = jnp.pad(p["pred"]["b2"], (0, OUT - 1))
    o = _mm(h1, W2, b=b2, post="sigmoid")
    return loss, o[:, :1]
